# baseline (speedup 1.0000x reference)
import functools

import jax
import jax.numpy as jnp
from jax import lax
from jax.experimental import pallas as pl
from jax.experimental.pallas import tpu as pltpu
from jax.experimental.pallas import tpu_sc as plsc

N = 8192
D = 256
M = 8192
G = 3
OUTW = 2 * D + 4
TAILW = D + 4

NW = 32
HALF = M // 2
GP_W = HALF // NW
CH = 32
NCH = GP_W // CH
IDX = CH * G
NGB = 3


def _tmpl_body(h_ref, o_ref):
    mean = jnp.sum(h_ref[...], axis=0, keepdims=True) * (1.0 / N)
    col4 = lax.broadcasted_iota(jnp.int32, (1, 4), 1)
    tail = jnp.where(col4 == 0, jnp.float32(G / 3.0), jnp.float32(0.0))
    row = jnp.concatenate([mean, tail], axis=1)
    o_ref[...] = jnp.broadcast_to(row, (CH, TAILW))


def _col_mean_tmpl(h):
    return pl.pallas_call(
        _tmpl_body,
        out_shape=jax.ShapeDtypeStruct((CH, TAILW), jnp.float32),
    )(h)


_mesh = plsc.VectorSubcoreMesh(core_axis_name="c", subcore_axis_name="s")


def _make_sc_build(half):
    @functools.partial(
        pl.kernel,
        mesh=_mesh,
        out_type=jax.ShapeDtypeStruct((HALF, OUTW), jnp.float32),
        scratch_types=[
            pltpu.VMEM((GP_W * G,), jnp.int32),
            pltpu.VMEM((IDX, D), jnp.float32),
            pltpu.VMEM((IDX, D), jnp.float32),
            pltpu.VMEM((IDX, D), jnp.float32),
            pltpu.VMEM((CH, OUTW), jnp.float32),
            pltpu.VMEM((CH, OUTW), jnp.float32),
            pltpu.SemaphoreType.DMA,
            pltpu.SemaphoreType.DMA,
            pltpu.SemaphoreType.DMA,
            pltpu.SemaphoreType.DMA,
            pltpu.SemaphoreType.DMA,
        ],
    )
    def _sc_build(h_hbm, gflat_hbm, tmpl_hbm, out_hbm,
                  idx_v, rows0, rows1, rows2, slab0, slab1,
                  sg0, sg1, sg2, so0, so1):
        cid = lax.axis_index("c")
        sid = lax.axis_index("s")
        wid = sid * 2 + cid
        base_g = half * HALF + wid * GP_W
        out_g = wid * GP_W

        rows = (rows0, rows1, rows2)
        slabs = (slab0, slab1)
        gsems = (sg0, sg1, sg2)
        osems = (so0, so1)

        pltpu.sync_copy(gflat_hbm.at[pl.ds(base_g * G, GP_W * G)], idx_v)

        def start_gather(k):
            b = k % NGB
            return pltpu.async_copy(
                h_hbm.at[idx_v.at[pl.ds(k * IDX, IDX)]], rows[b], gsems[b])

        ghandles = [None] * NGB
        for k in range(NGB - 1):
            ghandles[k] = start_gather(k)

        tail_handles = [
            pltpu.async_copy(tmpl_hbm, slabs[b].at[:, pl.ds(D, TAILW)], osems[b])
            for b in range(2)
        ]

        def pool(k):
            rows_v, slab_v = rows[k % NGB], slabs[k % 2]

            def body(g, carry):
                r = g * G
                for halfc in range(2):
                    cs = [halfc * 8 + c for c in range(8)]
                    a = [rows_v[r, pl.ds(c * 16, 16)] for c in cs]
                    b2 = [rows_v[r + 1, pl.ds(c * 16, 16)] for c in cs]
                    d2 = [rows_v[r + 2, pl.ds(c * 16, 16)] for c in cs]
                    for i, c in enumerate(cs):
                        slab_v[g, pl.ds(c * 16, 16)] = (
                            (a[i] + b2[i] + d2[i]) * jnp.float32(1.0 / G))
                return carry

            lax.fori_loop(0, CH, body, 0)

        def start_out(k):
            g0 = out_g + k * CH
            return pltpu.async_copy(
                slabs[k % 2], out_hbm.at[pl.ds(g0, CH), :], osems[k % 2])

        ohandles = list(tail_handles)
        for k in range(NCH):
            if k + NGB - 1 < NCH:
                ghandles[(k + NGB - 1) % NGB] = start_gather(k + NGB - 1)
            ghandles[k % NGB].wait()
            if ohandles[k % 2] is not None:
                ohandles[k % 2].wait()
            pool(k)
            ohandles[k % 2] = start_out(k)
        for b in range(2):
            if ohandles[b] is not None:
                ohandles[b].wait()

    return _sc_build


_sc_build_lo = _make_sc_build(0)
_sc_build_hi = _make_sc_build(1)


def kernel(h, groups):
    gflat = groups.astype(jnp.int32).reshape(-1)
    tmpl = _col_mean_tmpl(h)
    x_lo = _sc_build_lo(h, gflat, tmpl)
    x_hi = _sc_build_hi(h, gflat, tmpl)
    return jnp.concatenate([x_lo, x_hi], axis=0)


# speedup vs baseline: 1.3431x; 1.3431x over previous
import functools

import jax
import jax.numpy as jnp
from jax import lax
from jax.experimental import pallas as pl
from jax.experimental.pallas import tpu as pltpu
from jax.experimental.pallas import tpu_sc as plsc

N = 8192
D = 256
M = 8192
G = 3
OUTW = 2 * D + 4
TAILW = D + 4

NW = 32
GP_W = M // NW
CH = 32
NCH = GP_W // CH
IDX = CH * G
NGB = 3


def _tmpl_body(h_ref, o_ref):
    mean = jnp.sum(h_ref[...], axis=0, keepdims=True) * (1.0 / N)
    col4 = lax.broadcasted_iota(jnp.int32, (1, 4), 1)
    tail = jnp.where(col4 == 0, jnp.float32(G / 3.0), jnp.float32(0.0))
    row = jnp.concatenate([mean, tail], axis=1)
    o_ref[...] = jnp.broadcast_to(row, (CH, TAILW))


def _col_mean_tmpl(h):
    return pl.pallas_call(
        _tmpl_body,
        out_shape=jax.ShapeDtypeStruct((CH, TAILW), jnp.float32),
    )(h)


_mesh = plsc.VectorSubcoreMesh(core_axis_name="c", subcore_axis_name="s")


@functools.partial(
    pl.kernel,
    mesh=_mesh,
    out_type=jax.ShapeDtypeStruct((M, OUTW), jnp.float32),
    scratch_types=[
        pltpu.VMEM((GP_W * G,), jnp.int32),
        pltpu.VMEM((IDX, D), jnp.float32),
        pltpu.VMEM((IDX, D), jnp.float32),
        pltpu.VMEM((IDX, D), jnp.float32),
        pltpu.VMEM((CH, OUTW), jnp.float32),
        pltpu.VMEM((CH, OUTW), jnp.float32),
        pltpu.SemaphoreType.DMA,
        pltpu.SemaphoreType.DMA,
        pltpu.SemaphoreType.DMA,
        pltpu.SemaphoreType.DMA,
        pltpu.SemaphoreType.DMA,
    ],
)
def _sc_build(h_hbm, gflat_hbm, tmpl_hbm, out_hbm,
              idx_v, rows0, rows1, rows2, slab0, slab1,
              sg0, sg1, sg2, so0, so1):
    cid = lax.axis_index("c")
    sid = lax.axis_index("s")
    wid = sid * 2 + cid
    base_g = wid * GP_W

    rows = (rows0, rows1, rows2)
    slabs = (slab0, slab1)
    gsems = (sg0, sg1, sg2)
    osems = (so0, so1)

    pltpu.sync_copy(gflat_hbm.at[pl.ds(base_g * G, GP_W * G)], idx_v)

    def start_gather(k):
        b = k % NGB
        return pltpu.async_copy(
            h_hbm.at[idx_v.at[pl.ds(k * IDX, IDX)]], rows[b], gsems[b])

    ghandles = [None] * NGB
    for k in range(NGB - 1):
        ghandles[k] = start_gather(k)

    tail_handles = [
        pltpu.async_copy(tmpl_hbm, slabs[b].at[:, pl.ds(D, TAILW)], osems[b])
        for b in range(2)
    ]

    def pool(k):
        rows_v, slab_v = rows[k % NGB], slabs[k % 2]

        def body(g, carry):
            r = g * G
            for half in range(2):
                cs = [half * 8 + c for c in range(8)]
                a = [rows_v[r, pl.ds(c * 16, 16)] for c in cs]
                b2 = [rows_v[r + 1, pl.ds(c * 16, 16)] for c in cs]
                d2 = [rows_v[r + 2, pl.ds(c * 16, 16)] for c in cs]
                for i, c in enumerate(cs):
                    slab_v[g, pl.ds(c * 16, 16)] = (
                        (a[i] + b2[i] + d2[i]) * jnp.float32(1.0 / G))
            return carry

        lax.fori_loop(0, CH, body, 0)

    def start_out(k):
        g0 = base_g + k * CH
        return pltpu.async_copy(
            slabs[k % 2], out_hbm.at[pl.ds(g0, CH), :], osems[k % 2])

    ohandles = list(tail_handles)
    for k in range(NCH):
        if k + NGB - 1 < NCH:
            ghandles[(k + NGB - 1) % NGB] = start_gather(k + NGB - 1)
        ghandles[k % NGB].wait()
        if ohandles[k % 2] is not None:
            ohandles[k % 2].wait()
        pool(k)
        ohandles[k % 2] = start_out(k)
    for b in range(2):
        if ohandles[b] is not None:
            ohandles[b].wait()


def kernel(h, groups):
    gflat = groups.astype(jnp.int32).reshape(-1)
    tmpl = _col_mean_tmpl(h)
    return _sc_build(h, gflat, tmpl)
